# baseline (speedup 1.0000x reference)
import functools

import jax
import jax.numpy as jnp
import numpy as np
from jax import lax
from jax.experimental import pallas as pl
from jax.experimental.pallas import tpu as pltpu
from jax.experimental.pallas import tpu_sc as plsc

_NC = 2
_NS = 16
_NW = _NC * _NS
_L = 16


@functools.partial(jax.jit, static_argnames=("chunk",))
def _fixed_grid_bin(t, times, *, chunk=8192):
    n = t.shape[0]
    nt = times.shape[0]
    per_w = n // _NW
    n_chunks = per_w // chunk
    assert per_w * _NW == n and n_chunks * chunk == per_w and n_chunks >= 2
    scale = np.float32(nt - 1)
    max_bin = np.int32(nt - 2)

    mesh = plsc.VectorSubcoreMesh(
        core_axis_name="c", subcore_axis_name="s",
        num_cores=_NC, num_subcores=_NS)

    def body(t_hbm, times_hbm, ind_hbm, dt_hbm, dtind_hbm, tau_hbm, taun_hbm,
             times_v, dt_v, t_v, ind_v, dtind_v, tau_v, taun_v,
             in_sems, out_sems):
        wid = lax.axis_index("s") * _NC + lax.axis_index("c")
        pltpu.sync_copy(times_hbm, times_v)

        @pl.when(wid == 0)
        def _():
            for k in range((nt - 1) // _L):
                idx = lax.iota(jnp.int32, _L) + k * _L
                lo = plsc.load_gather(times_v, [idx])
                hi = plsc.load_gather(times_v, [idx + 1])
                dt_v[pl.ds(k * _L, _L)] = hi - lo
            pltpu.sync_copy(dt_v, dt_hbm)

        base_w = wid * per_w

        def in_copy(c, b):
            return pltpu.make_async_copy(
                t_hbm.at[pl.ds(base_w + c * chunk, chunk)], t_v[b], in_sems[b])

        def out_copies(c, b):
            base = base_w + c * chunk
            sl = pl.ds(base, chunk)
            return (
                pltpu.make_async_copy(ind_v[b], ind_hbm.at[sl], out_sems[b]),
                pltpu.make_async_copy(dtind_v[b], dtind_hbm.at[sl], out_sems[b]),
                pltpu.make_async_copy(tau_v[b], tau_hbm.at[sl], out_sems[b]),
                pltpu.make_async_copy(taun_v[b], taun_hbm.at[sl], out_sems[b]),
            )

        def compute(b):
            @pl.loop(0, chunk // _L, unroll=8)
            def _vec(i):
                s = i * _L
                v = t_v[b][pl.ds(s, _L)]
                bi = jnp.clip((v * scale).astype(jnp.int32), 0, max_bin)
                tau = plsc.load_gather(times_v, [bi])
                taun = plsc.load_gather(times_v, [bi + 1])
                ind_v[b][pl.ds(s, _L)] = bi
                tau_v[b][pl.ds(s, _L)] = tau
                taun_v[b][pl.ds(s, _L)] = taun
                dtind_v[b][pl.ds(s, _L)] = taun - tau

        in_copy(0, 0).start()

        @pl.loop(0, n_chunks, step=2)
        def _outer(c0):
            for b in range(2):
                c = c0 + b
                in_copy(c, b).wait()

                @pl.when(c + 1 < n_chunks)
                def _():
                    in_copy(c + 1, 1 - b).start()

                @pl.when(c >= 2)
                def _():
                    for cp in out_copies(c - 2, b):
                        cp.wait()

                compute(b)
                for cp in out_copies(c, b):
                    cp.start()

        for cp in out_copies(n_chunks - 2, 0):
            cp.wait()
        for cp in out_copies(n_chunks - 1, 1):
            cp.wait()

    return pl.kernel(
        body,
        out_type=(
            jax.ShapeDtypeStruct((n,), jnp.int32),
            jax.ShapeDtypeStruct((nt - 1,), jnp.float32),
            jax.ShapeDtypeStruct((n,), jnp.float32),
            jax.ShapeDtypeStruct((n,), jnp.float32),
            jax.ShapeDtypeStruct((n,), jnp.float32),
        ),
        mesh=mesh,
        compiler_params=pltpu.CompilerParams(needs_layout_passes=False),
        scratch_types=(
            pltpu.VMEM((nt,), jnp.float32),
            pltpu.VMEM((nt - 1,), jnp.float32),
            tuple(pltpu.VMEM((chunk,), jnp.float32) for _ in range(2)),
            tuple(pltpu.VMEM((chunk,), jnp.int32) for _ in range(2)),
            tuple(pltpu.VMEM((chunk,), jnp.float32) for _ in range(2)),
            tuple(pltpu.VMEM((chunk,), jnp.float32) for _ in range(2)),
            tuple(pltpu.VMEM((chunk,), jnp.float32) for _ in range(2)),
            tuple(pltpu.SemaphoreType.DMA for _ in range(2)),
            tuple(pltpu.SemaphoreType.DMA for _ in range(2)),
        ),
    )(t, times)


def kernel(t, z, discretization_times):
    ind, dt, dt_ind, tau_ind, tau_next_ind = _fixed_grid_bin(
        t, discretization_times)
    return (ind, dt, dt_ind, tau_ind, tau_next_ind, z)


# speedup vs baseline: 2.7873x; 2.7873x over previous
import functools

import jax
import jax.numpy as jnp
import numpy as np
from jax import lax
from jax.experimental import pallas as pl
from jax.experimental.pallas import tpu as pltpu
from jax.experimental.pallas import tpu_sc as plsc

_NC = 2
_NS = 16
_NW = _NC * _NS
_L = 16


@functools.partial(jax.jit, static_argnames=("chunk",))
def _fixed_grid_bin(t, times, *, chunk=8192):
    n = t.shape[0]
    nt = times.shape[0]
    per_w = n // _NW
    n_chunks = per_w // chunk
    assert per_w * _NW == n and n_chunks * chunk == per_w and n_chunks >= 2
    scale = np.float32(nt - 1)
    max_bin = np.int32(nt - 2)

    mesh = plsc.VectorSubcoreMesh(
        core_axis_name="c", subcore_axis_name="s",
        num_cores=_NC, num_subcores=_NS)

    def body(t_hbm, times_hbm, ind_hbm, dt_hbm, dtind_hbm, tau_hbm, taun_hbm,
             times_v, dt_v, t_v, ind_v, dtind_v, tau_v, taun_v,
             in_sems, out_sems):
        wid = lax.axis_index("s") * _NC + lax.axis_index("c")
        pltpu.sync_copy(times_hbm, times_v)

        @pl.when(wid == 0)
        def _():
            for k in range((nt - 1) // _L):
                idx = lax.iota(jnp.int32, _L) + k * _L
                lo = plsc.load_gather(times_v, [idx])
                hi = plsc.load_gather(times_v, [idx + 1])
                dt_v[pl.ds(k * _L, _L)] = hi - lo
            pltpu.sync_copy(dt_v, dt_hbm)

        base_w = wid * per_w

        def in_copy(c, b):
            return pltpu.make_async_copy(
                t_hbm.at[pl.ds(base_w + c * chunk, chunk)], t_v[b], in_sems[b])

        def out_copies(c, b):
            base = base_w + c * chunk
            sl = pl.ds(base, chunk)
            return (
                pltpu.make_async_copy(ind_v[b], ind_hbm.at[sl], out_sems[b]),
                pltpu.make_async_copy(dtind_v[b], dtind_hbm.at[sl], out_sems[b]),
                pltpu.make_async_copy(tau_v[b], tau_hbm.at[sl], out_sems[b]),
                pltpu.make_async_copy(taun_v[b], taun_hbm.at[sl], out_sems[b]),
            )

        def compute(b):
            @plsc.parallel_loop(0, chunk // _L, unroll=4)
            def _vec(i):
                s = i * _L
                v = t_v[b][pl.ds(s, _L)]
                bi = jnp.clip((v * scale).astype(jnp.int32), 0, max_bin)
                tau = plsc.load_gather(times_v, [bi])
                taun = plsc.load_gather(times_v, [bi + 1])
                ind_v[b][pl.ds(s, _L)] = bi
                tau_v[b][pl.ds(s, _L)] = tau
                taun_v[b][pl.ds(s, _L)] = taun
                dtind_v[b][pl.ds(s, _L)] = taun - tau

        in_copy(0, 0).start()

        @pl.loop(0, n_chunks, step=2)
        def _outer(c0):
            for b in range(2):
                c = c0 + b
                in_copy(c, b).wait()

                @pl.when(c + 1 < n_chunks)
                def _():
                    in_copy(c + 1, 1 - b).start()

                @pl.when(c >= 2)
                def _():
                    for cp in out_copies(c - 2, b):
                        cp.wait()

                compute(b)
                for cp in out_copies(c, b):
                    cp.start()

        for cp in out_copies(n_chunks - 2, 0):
            cp.wait()
        for cp in out_copies(n_chunks - 1, 1):
            cp.wait()

    return pl.kernel(
        body,
        out_type=(
            jax.ShapeDtypeStruct((n,), jnp.int32),
            jax.ShapeDtypeStruct((nt - 1,), jnp.float32),
            jax.ShapeDtypeStruct((n,), jnp.float32),
            jax.ShapeDtypeStruct((n,), jnp.float32),
            jax.ShapeDtypeStruct((n,), jnp.float32),
        ),
        mesh=mesh,
        compiler_params=pltpu.CompilerParams(needs_layout_passes=False),
        scratch_types=(
            pltpu.VMEM((nt,), jnp.float32),
            pltpu.VMEM((nt - 1,), jnp.float32),
            tuple(pltpu.VMEM((chunk,), jnp.float32) for _ in range(2)),
            tuple(pltpu.VMEM((chunk,), jnp.int32) for _ in range(2)),
            tuple(pltpu.VMEM((chunk,), jnp.float32) for _ in range(2)),
            tuple(pltpu.VMEM((chunk,), jnp.float32) for _ in range(2)),
            tuple(pltpu.VMEM((chunk,), jnp.float32) for _ in range(2)),
            tuple(pltpu.SemaphoreType.DMA for _ in range(2)),
            tuple(pltpu.SemaphoreType.DMA for _ in range(2)),
        ),
    )(t, times)


def kernel(t, z, discretization_times):
    ind, dt, dt_ind, tau_ind, tau_next_ind = _fixed_grid_bin(
        t, discretization_times)
    return (ind, dt, dt_ind, tau_ind, tau_next_ind, z)


# speedup vs baseline: 2.8046x; 1.0062x over previous
import functools

import jax
import jax.numpy as jnp
import numpy as np
from jax import lax
from jax.experimental import pallas as pl
from jax.experimental.pallas import tpu as pltpu
from jax.experimental.pallas import tpu_sc as plsc

_NC = 2
_NS = 16
_NW = _NC * _NS
_L = 16


@functools.partial(jax.jit, static_argnames=("chunk",))
def _fixed_grid_bin(t, times, *, chunk=8192):
    n = t.shape[0]
    nt = times.shape[0]
    per_w = n // _NW
    n_chunks = per_w // chunk
    assert per_w * _NW == n and n_chunks * chunk == per_w and n_chunks >= 2
    scale = np.float32(nt - 1)
    max_bin = np.int32(nt - 2)

    mesh = plsc.VectorSubcoreMesh(
        core_axis_name="c", subcore_axis_name="s",
        num_cores=_NC, num_subcores=_NS)

    def body(t_hbm, times_hbm, ind_hbm, dt_hbm, dtind_hbm, tau_hbm, taun_hbm,
             times_v, dt_v, t_v, ind_v, dtind_v, tau_v, taun_v,
             in_sems, out_sems):
        wid = lax.axis_index("s") * _NC + lax.axis_index("c")
        pltpu.sync_copy(times_hbm, times_v)

        @pl.when(wid == 0)
        def _():
            for k in range((nt - 1) // _L):
                idx = lax.iota(jnp.int32, _L) + k * _L
                lo = plsc.load_gather(times_v, [idx])
                hi = plsc.load_gather(times_v, [idx + 1])
                dt_v[pl.ds(k * _L, _L)] = hi - lo
            pltpu.sync_copy(dt_v, dt_hbm)

        base_w = wid * per_w

        def in_copy(c, b):
            return pltpu.make_async_copy(
                t_hbm.at[pl.ds(base_w + c * chunk, chunk)], t_v[b], in_sems[b])

        def out_copies(c, b):
            base = base_w + c * chunk
            sl = pl.ds(base, chunk)
            return (
                pltpu.make_async_copy(ind_v[b], ind_hbm.at[sl], out_sems[b]),
                pltpu.make_async_copy(dtind_v[b], dtind_hbm.at[sl], out_sems[b]),
                pltpu.make_async_copy(tau_v[b], tau_hbm.at[sl], out_sems[b]),
                pltpu.make_async_copy(taun_v[b], taun_hbm.at[sl], out_sems[b]),
            )

        def compute(b):
            @plsc.parallel_loop(0, chunk // _L, unroll=8)
            def _vec(i):
                s = i * _L
                v = t_v[b][pl.ds(s, _L)]
                bi = jnp.clip((v * scale).astype(jnp.int32), 0, max_bin)
                tau = plsc.load_gather(times_v, [bi])
                taun = plsc.load_gather(times_v, [bi + 1])
                ind_v[b][pl.ds(s, _L)] = bi
                tau_v[b][pl.ds(s, _L)] = tau
                taun_v[b][pl.ds(s, _L)] = taun
                dtind_v[b][pl.ds(s, _L)] = taun - tau

        in_copy(0, 0).start()

        @pl.loop(0, n_chunks, step=2)
        def _outer(c0):
            for b in range(2):
                c = c0 + b
                in_copy(c, b).wait()

                @pl.when(c + 1 < n_chunks)
                def _():
                    in_copy(c + 1, 1 - b).start()

                @pl.when(c >= 2)
                def _():
                    for cp in out_copies(c - 2, b):
                        cp.wait()

                compute(b)
                for cp in out_copies(c, b):
                    cp.start()

        for cp in out_copies(n_chunks - 2, 0):
            cp.wait()
        for cp in out_copies(n_chunks - 1, 1):
            cp.wait()

    return pl.kernel(
        body,
        out_type=(
            jax.ShapeDtypeStruct((n,), jnp.int32),
            jax.ShapeDtypeStruct((nt - 1,), jnp.float32),
            jax.ShapeDtypeStruct((n,), jnp.float32),
            jax.ShapeDtypeStruct((n,), jnp.float32),
            jax.ShapeDtypeStruct((n,), jnp.float32),
        ),
        mesh=mesh,
        compiler_params=pltpu.CompilerParams(needs_layout_passes=False),
        scratch_types=(
            pltpu.VMEM((nt,), jnp.float32),
            pltpu.VMEM((nt - 1,), jnp.float32),
            tuple(pltpu.VMEM((chunk,), jnp.float32) for _ in range(2)),
            tuple(pltpu.VMEM((chunk,), jnp.int32) for _ in range(2)),
            tuple(pltpu.VMEM((chunk,), jnp.float32) for _ in range(2)),
            tuple(pltpu.VMEM((chunk,), jnp.float32) for _ in range(2)),
            tuple(pltpu.VMEM((chunk,), jnp.float32) for _ in range(2)),
            tuple(pltpu.SemaphoreType.DMA for _ in range(2)),
            tuple(pltpu.SemaphoreType.DMA for _ in range(2)),
        ),
    )(t, times)


def kernel(t, z, discretization_times):
    ind, dt, dt_ind, tau_ind, tau_next_ind = _fixed_grid_bin(
        t, discretization_times)
    return (ind, dt, dt_ind, tau_ind, tau_next_ind, z)
